# baseline (speedup 1.0000x reference)
import functools

import jax
import jax.numpy as jnp
from jax import lax
from jax.experimental import pallas as pl
from jax.experimental.pallas import tpu as pltpu
from jax.experimental.pallas import tpu_sc as plsc

E = 8
T = 2048
H = 1024
I = 1024
B = 256
PAD_T = 4096 + 8 * B
NB = PAD_T // B

NC, NS = 2, 16
NW = NC * NS
W = PAD_T // NW
G = 48
NG = W // G
TPW = T // NW
_SC_MESH = plsc.VectorSubcoreMesh(core_axis_name="c", subcore_axis_name="s")


def _routing_body(l_ref, pos0_ref, pos1_ref, w0_ref, w1_ref, eid_ref, nblk_ref):
    l = l_ref[...]
    ei = lax.broadcasted_iota(jnp.int32, (T, E), 1)
    m1 = jnp.max(l, axis=1, keepdims=True)
    a1 = jnp.min(jnp.where(l == m1, ei, E), axis=1, keepdims=True)
    l2 = jnp.where(ei == a1, -jnp.inf, l)
    m2 = jnp.max(l2, axis=1, keepdims=True)
    a2 = jnp.min(jnp.where(l2 == m2, ei, E), axis=1, keepdims=True)
    w0 = jax.nn.sigmoid(m1 - m2)

    oh1 = ei == a1
    oh2 = ei == a2
    C = oh1.astype(jnp.int32) + oh2.astype(jnp.int32)
    inc = C
    s = 1
    while s < T:
        inc = inc + jnp.concatenate(
            [jnp.zeros((s, E), jnp.int32), inc[:-s]], axis=0)
        s *= 2
    P = inc - C
    counts = lax.slice(inc, (T - 1, 0), (T, E))
    padded = ((counts + (B - 1)) // B) * B
    pinc = padded
    s = 1
    while s < E:
        pinc = pinc + jnp.concatenate(
            [jnp.zeros((1, s), jnp.int32), pinc[:, :-s]], axis=1)
        s *= 2
    poff = pinc - padded

    pos0_ref[...] = jnp.sum(jnp.where(oh1, poff + P, 0), axis=1,
                            keepdims=True).reshape(1, T)
    pos1_ref[...] = jnp.sum(jnp.where(oh2, poff + P, 0), axis=1,
                            keepdims=True).reshape(1, T)
    w0_ref[...] = w0.reshape(1, T)
    w1_ref[...] = (1.0 - w0).reshape(1, T)

    gb = lax.broadcasted_iota(jnp.int32, (1, NB), 1) * B
    acc = jnp.zeros((1, NB), jnp.int32)
    for e in range(E):
        pe = lax.slice(poff, (0, e), (1, e + 1))
        acc = acc + (pe <= gb).astype(jnp.int32)
    eid_ref[...] = acc - 1
    nblk_ref[...] = jnp.sum(padded, keepdims=True)[:, :1] // B


def _routing(router_logits):
    return pl.pallas_call(
        _routing_body,
        out_shape=[
            jax.ShapeDtypeStruct((1, T), jnp.int32),
            jax.ShapeDtypeStruct((1, T), jnp.int32),
            jax.ShapeDtypeStruct((1, T), jnp.float32),
            jax.ShapeDtypeStruct((1, T), jnp.float32),
            jax.ShapeDtypeStruct((1, NB), jnp.int32),
            jax.ShapeDtypeStruct((1, 1), jnp.int32),
        ],
    )(router_logits)


def _ffn_body(eid_ref, nblk_ref, x_ref, w13_ref, w2_ref, ws_ref, y_ref):
    g = pl.program_id(0)

    @pl.when(g < nblk_ref[0])
    def _():
        x = x_ref[...]
        gu = lax.dot_general(x, w13_ref[0], (((1,), (1,)), ((), ())),
                             preferred_element_type=jnp.float32)
        gate = gu[:, :I]
        up = gu[:, I:]
        h = gate * jax.nn.sigmoid(gate) * up
        y = lax.dot_general(h, w2_ref[0], (((1,), (1,)), ((), ())),
                            preferred_element_type=jnp.float32)
        y_ref[...] = y * ws_ref[0, 0][:, None]


def _ffn(eid, nblk, x_sorted, w13, w2, w_sorted):
    ws3 = w_sorted.reshape(NB, 1, B)
    spec = pltpu.PrefetchScalarGridSpec(
        num_scalar_prefetch=2,
        grid=(NB,),
        in_specs=[
            pl.BlockSpec((B, H), lambda g, eid, nb: (g, 0)),
            pl.BlockSpec((1, 2 * I, H), lambda g, eid, nb: (eid[g], 0, 0)),
            pl.BlockSpec((1, H, I), lambda g, eid, nb: (eid[g], 0, 0)),
            pl.BlockSpec((1, 1, B), lambda g, eid, nb: (g, 0, 0)),
        ],
        out_specs=pl.BlockSpec((B, H), lambda g, eid, nb: (g, 0)),
    )
    return pl.pallas_call(
        _ffn_body,
        grid_spec=spec,
        out_shape=jax.ShapeDtypeStruct((PAD_T, H), jnp.float32),
    )(eid, nblk, x_sorted, w13, w2, ws3)


@functools.partial(
    pl.kernel,
    mesh=_SC_MESH,
    compiler_params=pltpu.CompilerParams(needs_layout_passes=False),
    out_type=[
        jax.ShapeDtypeStruct((PAD_T,), jnp.int32),
        jax.ShapeDtypeStruct((PAD_T,), jnp.float32),
    ],
    scratch_types=[
        pltpu.VMEM((TPW,), jnp.int32),
        pltpu.VMEM((TPW,), jnp.int32),
        pltpu.VMEM((TPW,), jnp.float32),
        pltpu.VMEM((TPW,), jnp.float32),
        pltpu.VMEM((TPW,), jnp.int32),
        pltpu.VMEM((TPW,), jnp.int32),
        pltpu.SemaphoreType.DMA,
        pltpu.SemaphoreType.DMA,
        pltpu.SemaphoreType.DMA,
        pltpu.SemaphoreType.DMA,
    ],
)
def _sc_pair_scatter(pos0_hbm, pos1_hbm, w0_hbm, w1_hbm,
                     key2_hbm, ws_hbm,
                     p0_v, p1_v, w0_v, w1_v, t0_v, t1_v, s0, s1, s2, s3):
    wid = lax.axis_index("s") * NC + lax.axis_index("c")
    tb = wid * TPW
    pltpu.sync_copy(pos0_hbm.at[pl.ds(tb, TPW)], p0_v)
    pltpu.sync_copy(pos1_hbm.at[pl.ds(tb, TPW)], p1_v)
    pltpu.sync_copy(w0_hbm.at[pl.ds(tb, TPW)], w0_v)
    pltpu.sync_copy(w1_hbm.at[pl.ds(tb, TPW)], w1_v)

    iota16 = lax.iota(jnp.int32, 16)
    for i in range(TPW // 16):
        tok2 = (tb + i * 16 + iota16) * 2
        t0_v[pl.ds(i * 16, 16)] = tok2 + 1
        t1_v[pl.ds(i * 16, 16)] = tok2 + 2

    d0 = pltpu.async_copy(t0_v, key2_hbm.at[p0_v], s0)
    d1 = pltpu.async_copy(t1_v, key2_hbm.at[p1_v], s1)
    d2 = pltpu.async_copy(w0_v, ws_hbm.at[p0_v], s2)
    d3 = pltpu.async_copy(w1_v, ws_hbm.at[p1_v], s3)
    d0.wait()
    d1.wait()
    d2.wait()
    d3.wait()


@functools.partial(
    pl.kernel,
    mesh=_SC_MESH,
    compiler_params=pltpu.CompilerParams(needs_layout_passes=False),
    out_type=jax.ShapeDtypeStruct((PAD_T, H), jnp.float32),
    scratch_types=[
        pltpu.VMEM((W,), jnp.int32),
        pltpu.VMEM((W,), jnp.int32),
        pltpu.VMEM((G, H), jnp.float32),
        pltpu.VMEM((G, H), jnp.float32),
        pltpu.SemaphoreType.DMA,
        pltpu.SemaphoreType.DMA,
        pltpu.SemaphoreType.DMA,
        pltpu.SemaphoreType.DMA,
    ],
)
def _sc_row_gather(key2_hbm, hidden_hbm, xs_hbm,
                   k2_v, tid_v, buf0, buf1, g0, g1, e0, e1):
    wid = lax.axis_index("s") * NC + lax.axis_index("c")
    base = wid * W
    pltpu.sync_copy(key2_hbm.at[pl.ds(base, W)], k2_v)

    iota16 = lax.iota(jnp.int32, 16)
    for i in range(W // 16):
        sl = pl.ds(i * 16, 16)
        k2 = k2_v[sl]
        spread = (base + i * 16 + iota16) & (T - 1)
        tid = jnp.where(k2 == 0, spread,
                        jnp.minimum(jnp.maximum((k2 - 1) >> 1, 0), T - 1))
        tid_v[sl] = tid

    bufs = (buf0, buf1)
    gsems = (g0, g1)
    esems = (e0, e1)
    writes = [None, None]
    for c in range(NG):
        b = c % 2
        if writes[b] is not None:
            writes[b].wait()
        d = pltpu.async_copy(hidden_hbm.at[tid_v.at[pl.ds(c * G, G)]],
                             bufs[b], gsems[b])
        d.wait()
        writes[b] = pltpu.async_copy(bufs[b], xs_hbm.at[pl.ds(base + c * G, G)],
                                     esems[b])
    writes[0].wait()
    writes[1].wait()


_CTOK = TPW // 2


@functools.partial(
    pl.kernel,
    mesh=_SC_MESH,
    compiler_params=pltpu.CompilerParams(needs_layout_passes=False),
    out_type=jax.ShapeDtypeStruct((T, H), jnp.float32),
    scratch_types=[
        pltpu.VMEM((TPW,), jnp.int32),
        pltpu.VMEM((TPW,), jnp.int32),
        pltpu.VMEM((_CTOK, H), jnp.float32),
        pltpu.VMEM((_CTOK, H), jnp.float32),
        pltpu.SemaphoreType.DMA,
    ],
)
def _sc_combine(pos0_hbm, pos1_hbm, y_hbm, out_hbm,
                p0_v, p1_v, buf_v, acc_v, sem):
    wid = lax.axis_index("s") * NC + lax.axis_index("c")
    base = wid * TPW
    pltpu.sync_copy(pos0_hbm.at[pl.ds(base, TPW)], p0_v)
    pltpu.sync_copy(pos1_hbm.at[pl.ds(base, TPW)], p1_v)

    for c in range(TPW // _CTOK):
        pltpu.async_copy(y_hbm.at[p0_v.at[pl.ds(c * _CTOK, _CTOK)]],
                         buf_v, sem).wait()
        pltpu.async_copy(y_hbm.at[p1_v.at[pl.ds(c * _CTOK, _CTOK)]],
                         acc_v, sem).wait()

        def addrow(r, cc):
            for j in range(H // 16):
                sl = pl.ds(j * 16, 16)
                acc_v[r, sl] = acc_v[r, sl] + buf_v[r, sl]
            return cc

        lax.fori_loop(0, _CTOK, addrow, 0)
        pltpu.sync_copy(acc_v, out_hbm.at[pl.ds(base + c * _CTOK, _CTOK)])


def kernel(hidden_states, router_logits, w13_weight, w2_weight):
    _ABL = 4
    pos0, pos1, w0, w1, eid, nblk = _routing(router_logits)
    pos0 = pos0.reshape(T)
    pos1 = pos1.reshape(T)
    if _ABL == 1:
        return hidden_states * w0.reshape(T, 1)

    key2, wso = _sc_pair_scatter(pos0, pos1, w0.reshape(T), w1.reshape(T))
    x_sorted = _sc_row_gather(key2, hidden_states)
    if _ABL == 2:
        return x_sorted[:T]

    y = _ffn(eid.reshape(NB), nblk.reshape(1), x_sorted,
             w13_weight, w2_weight, wso)
    if _ABL == 3:
        return y[:T]

    return _sc_combine(pos0, pos1, y)
